# Initial kernel scaffold; baseline (speedup 1.0000x reference)
import jax
import jax.numpy as jnp
from jax.experimental import pallas as pl


def kernel(x, nodes, fact, fact_dim, params, bias):
    raise NotImplementedError("write your pallas kernel here")


# speedup vs baseline: 2.0387x; 2.0387x over previous
import jax
import jax.numpy as jnp
from jax import lax
from jax.experimental import pallas as pl
from jax.experimental.pallas import tpu as pltpu

_MA = 13
_TS = 256


def _seg_mm_kernel(tile_r, id_r, ls_r, le_r, x0_ref, x1_ref, w_ref, b_ref,
                   o0_ref, o1_ref):
    i = pl.program_id(0)
    ls = ls_r[i]
    le = le_r[i]
    rows = lax.broadcasted_iota(jnp.int32, (_TS, 1), 0)
    m = (rows >= ls) & (rows < le)
    wmat = w_ref[0]
    bvec = b_ref[0, 0]
    y0 = jnp.maximum(
        jnp.dot(x0_ref[:], wmat, preferred_element_type=jnp.float32) + bvec, 0.0)
    y1 = jnp.maximum(
        jnp.dot(x1_ref[:], wmat, preferred_element_type=jnp.float32) + bvec, 0.0)
    o0_ref[:] = jnp.where(m, y0, o0_ref[:])
    o1_ref[:] = jnp.where(m, y1, o1_ref[:])


def kernel(x, nodes, fact, fact_dim, params, bias):
    del fact_dim
    F = fact.shape[0]
    L = nodes.shape[1]
    P = params.shape[0]
    R = params.shape[2]
    NT = (F + _TS - 1) // _TS
    Fpad = NT * _TS
    WMAX = P + NT

    ids = (x[fact[:, 0], 1] * _MA + x[fact[:, 0], 2]).astype(jnp.int32)
    ids_p = jnp.concatenate([ids, jnp.zeros((Fpad - F,), jnp.int32)])

    iota = jnp.arange(Fpad, dtype=jnp.int32)
    sorted_ids, order = lax.sort_key_val(ids_p, iota)
    prev = jnp.concatenate([jnp.full((1,), -1, jnp.int32), sorted_ids[:-1]])
    flags = (sorted_ids != prev) | (iota % _TS == 0)
    starts = jnp.sort(jnp.where(flags, iota, Fpad))[:WMAX]
    ends = jnp.concatenate([starts[1:], jnp.full((1,), Fpad, jnp.int32)])
    tile_w = jnp.minimum(starts // _TS, NT - 1).astype(jnp.int32)
    ls_w = (starts - tile_w * _TS).astype(jnp.int32)
    le_w = (jnp.minimum(ends, (tile_w + 1) * _TS) - tile_w * _TS).astype(jnp.int32)
    id_w = sorted_ids[jnp.minimum(starts, Fpad - 1)]

    zpad = jnp.zeros((Fpad - F,), fact.dtype)
    f0 = jnp.concatenate([fact[:, 0], zpad])
    f1 = jnp.concatenate([fact[:, 1], zpad])
    xs0 = nodes[f0[order]]
    xs1 = nodes[f1[order]]

    grid_spec = pltpu.PrefetchScalarGridSpec(
        num_scalar_prefetch=4,
        grid=(WMAX,),
        in_specs=[
            pl.BlockSpec((_TS, L), lambda w, t, i, s, e: (t[w], 0)),
            pl.BlockSpec((_TS, L), lambda w, t, i, s, e: (t[w], 0)),
            pl.BlockSpec((1, L, R), lambda w, t, i, s, e: (i[w], 0, 0)),
            pl.BlockSpec((1, 1, R), lambda w, t, i, s, e: (i[w], 0, 0)),
        ],
        out_specs=[
            pl.BlockSpec((_TS, R), lambda w, t, i, s, e: (t[w], 0)),
            pl.BlockSpec((_TS, R), lambda w, t, i, s, e: (t[w], 0)),
        ],
    )
    ys0, ys1 = pl.pallas_call(
        _seg_mm_kernel,
        grid_spec=grid_spec,
        out_shape=[jax.ShapeDtypeStruct((Fpad, R), jnp.float32)] * 2,
    )(tile_w, id_w, ls_w, le_w, xs0, xs1, params, bias)

    rank = lax.sort_key_val(order, iota)[1]
    out0 = ys0[rank][:F]
    out1 = ys1[rank][:F]
    return jnp.stack([out0, out1])


# speedup vs baseline: 2.1109x; 1.0354x over previous
import functools
import jax
import jax.numpy as jnp
from jax import lax
from jax.experimental import pallas as pl
from jax.experimental.pallas import tpu as pltpu
from jax.experimental.pallas import tpu_sc as plsc

_MA = 13
_TS = 256
_NW = 32
_CH = 80


def _seg_mm_kernel(tile_r, id_r, ls_r, le_r, x0_ref, x1_ref, w_ref, b_ref,
                   o0_ref, o1_ref):
    i = pl.program_id(0)
    ls = ls_r[i]
    le = le_r[i]
    rows = lax.broadcasted_iota(jnp.int32, (_TS, 1), 0)
    m = (rows >= ls) & (rows < le)
    wmat = w_ref[0]
    bvec = b_ref[0, 0]
    y0 = jnp.maximum(
        jnp.dot(x0_ref[:, :64], wmat, preferred_element_type=jnp.float32) + bvec,
        0.0)
    y1 = jnp.maximum(
        jnp.dot(x1_ref[:, :64], wmat, preferred_element_type=jnp.float32) + bvec,
        0.0)
    o0_ref[:] = jnp.where(m, y0, o0_ref[:])
    o1_ref[:] = jnp.where(m, y1, o1_ref[:])


def _wid():
    return lax.axis_index("s") * 2 + lax.axis_index("c")


def _make_ids_kernel(Fpad):
    E = Fpad // _NW
    NCH = E // _CH
    mesh = plsc.VectorSubcoreMesh(core_axis_name="c", subcore_axis_name="s")

    @functools.partial(
        pl.kernel, mesh=mesh,
        out_type=jax.ShapeDtypeStruct((Fpad,), jnp.int32),
        scratch_types=[
            pltpu.VMEM((_CH,), jnp.int32),
            pltpu.VMEM((_CH,), jnp.int32),
            pltpu.VMEM((_CH,), jnp.int32),
            pltpu.SemaphoreType.DMA,
        ],
    )
    def ids_kernel(x1_hbm, x2_hbm, f0_hbm, ids_hbm, fv, a1v, a2v, sem):
        base = _wid() * E
        for c in range(NCH):
            pltpu.sync_copy(f0_hbm.at[pl.ds(base + c * _CH, _CH)], fv)
            pltpu.async_copy(x1_hbm.at[fv], a1v, sem).wait()
            pltpu.async_copy(x2_hbm.at[fv], a2v, sem).wait()
            for k in range(_CH // 16):
                sl = pl.ds(k * 16, 16)
                fv[sl] = a1v[sl] * _MA + a2v[sl]
            pltpu.sync_copy(fv, ids_hbm.at[pl.ds(base + c * _CH, _CH)])

    return ids_kernel


def _make_gather_rows_kernel(N, Fpad):
    E = Fpad // _NW
    NCH = E // _CH
    mesh = plsc.VectorSubcoreMesh(core_axis_name="c", subcore_axis_name="s")

    @functools.partial(
        pl.kernel, mesh=mesh,
        out_type=[jax.ShapeDtypeStruct((Fpad, 128), jnp.float32)] * 2,
        scratch_types=[
            pltpu.VMEM((_CH,), jnp.int32),
            pltpu.VMEM((_CH,), jnp.int32),
            pltpu.VMEM((_CH,), jnp.int32),
            pltpu.VMEM((_CH, 128), jnp.float32),
            pltpu.VMEM((_CH, 128), jnp.float32),
            pltpu.SemaphoreType.DMA,
        ],
    )
    def gather_kernel(order_hbm, f0_hbm, f1_hbm, nodes_hbm, xs0_hbm, xs1_hbm,
                      ov, g0v, g1v, r0v, r1v, sem):
        base = _wid() * E
        for c in range(NCH):
            pltpu.sync_copy(order_hbm.at[pl.ds(base + c * _CH, _CH)], ov)
            pltpu.async_copy(f0_hbm.at[ov], g0v, sem).wait()
            pltpu.async_copy(f1_hbm.at[ov], g1v, sem).wait()
            pltpu.async_copy(nodes_hbm.at[g0v], r0v, sem).wait()
            pltpu.async_copy(nodes_hbm.at[g1v], r1v, sem).wait()
            pltpu.sync_copy(r0v, xs0_hbm.at[pl.ds(base + c * _CH, _CH)])
            pltpu.sync_copy(r1v, xs1_hbm.at[pl.ds(base + c * _CH, _CH)])

    return gather_kernel


def _make_unsort_kernel(R, Fpad):
    E = Fpad // _NW
    NCH = E // _CH
    mesh = plsc.VectorSubcoreMesh(core_axis_name="c", subcore_axis_name="s")

    @functools.partial(
        pl.kernel, mesh=mesh,
        out_type=[jax.ShapeDtypeStruct((Fpad, R), jnp.float32)] * 2,
        scratch_types=[
            pltpu.VMEM((_CH,), jnp.int32),
            pltpu.VMEM((_CH, R), jnp.float32),
            pltpu.VMEM((_CH, R), jnp.float32),
            pltpu.SemaphoreType.DMA,
        ],
    )
    def unsort_kernel(rank_hbm, ys0_hbm, ys1_hbm, out0_hbm, out1_hbm,
                      rv, r0v, r1v, sem):
        base = _wid() * E
        for c in range(NCH):
            pltpu.sync_copy(rank_hbm.at[pl.ds(base + c * _CH, _CH)], rv)
            pltpu.async_copy(ys0_hbm.at[rv], r0v, sem).wait()
            pltpu.async_copy(ys1_hbm.at[rv], r1v, sem).wait()
            pltpu.sync_copy(r0v, out0_hbm.at[pl.ds(base + c * _CH, _CH)])
            pltpu.sync_copy(r1v, out1_hbm.at[pl.ds(base + c * _CH, _CH)])

    return unsort_kernel


def kernel(x, nodes, fact, fact_dim, params, bias):
    del fact_dim
    F = fact.shape[0]
    N = nodes.shape[0]
    L = nodes.shape[1]
    P = params.shape[0]
    R = params.shape[2]
    NT = (F + _TS - 1) // _TS
    Fpad = NT * _TS
    assert Fpad % (_NW * _CH) == 0
    WMAX = P + NT

    zpad = jnp.zeros((Fpad - F,), jnp.int32)
    f0 = jnp.concatenate([fact[:, 0].astype(jnp.int32), zpad])
    f1 = jnp.concatenate([fact[:, 1].astype(jnp.int32), zpad])
    x1 = x[:, 1].astype(jnp.int32)
    x2 = x[:, 2].astype(jnp.int32)
    nodes_pad = jnp.concatenate(
        [nodes, jnp.zeros((N, 128 - L), jnp.float32)], axis=1)

    ids_p = _make_ids_kernel(Fpad)(x1, x2, f0)

    iota = jnp.arange(Fpad, dtype=jnp.int32)
    sorted_ids, order = lax.sort_key_val(ids_p, iota)
    prev = jnp.concatenate([jnp.full((1,), -1, jnp.int32), sorted_ids[:-1]])
    flags = (sorted_ids != prev) | (iota % _TS == 0)
    pos = jnp.cumsum(flags.astype(jnp.int32)) - 1
    starts = (jnp.full((WMAX,), Fpad, jnp.int32)
              .at[jnp.where(flags, pos, WMAX)].set(iota, mode="drop"))
    ends = jnp.concatenate([starts[1:], jnp.full((1,), Fpad, jnp.int32)])
    tile_w = jnp.minimum(starts // _TS, NT - 1).astype(jnp.int32)
    ls_w = (starts - tile_w * _TS).astype(jnp.int32)
    le_w = (jnp.minimum(ends, (tile_w + 1) * _TS) - tile_w * _TS).astype(jnp.int32)
    id_w = sorted_ids[jnp.minimum(starts, Fpad - 1)]
    rank = jnp.zeros((Fpad,), jnp.int32).at[order].set(iota)

    xs0, xs1 = _make_gather_rows_kernel(N, Fpad)(order, f0, f1, nodes_pad)

    grid_spec = pltpu.PrefetchScalarGridSpec(
        num_scalar_prefetch=4,
        grid=(WMAX,),
        in_specs=[
            pl.BlockSpec((_TS, 128), lambda w, t, i, s, e: (t[w], 0)),
            pl.BlockSpec((_TS, 128), lambda w, t, i, s, e: (t[w], 0)),
            pl.BlockSpec((1, L, R), lambda w, t, i, s, e: (i[w], 0, 0)),
            pl.BlockSpec((1, 1, R), lambda w, t, i, s, e: (i[w], 0, 0)),
        ],
        out_specs=[
            pl.BlockSpec((_TS, R), lambda w, t, i, s, e: (t[w], 0)),
            pl.BlockSpec((_TS, R), lambda w, t, i, s, e: (t[w], 0)),
        ],
    )
    ys0, ys1 = pl.pallas_call(
        _seg_mm_kernel,
        grid_spec=grid_spec,
        out_shape=[jax.ShapeDtypeStruct((Fpad, R), jnp.float32)] * 2,
    )(tile_w, id_w, ls_w, le_w, xs0, xs1, params, bias)

    out0, out1 = _make_unsort_kernel(R, Fpad)(rank, ys0, ys1)
    return jnp.stack([out0[:F], out1[:F]])


# speedup vs baseline: 2.2474x; 1.0647x over previous
import functools
import jax
import jax.numpy as jnp
from jax import lax
from jax.experimental import pallas as pl
from jax.experimental.pallas import tpu as pltpu
from jax.experimental.pallas import tpu_sc as plsc

_MA = 13
_TS = 256
_NW = 32


def _splits(total, cap=128):
    out, off = [], 0
    while off < total:
        sz = min(cap, total - off)
        out.append((off, sz))
        off += sz
    return out


def _seg_mm_kernel(tile_r, id_r, ls_r, le_r, x0_ref, x1_ref, w_ref, b_ref,
                   o0_ref, o1_ref):
    i = pl.program_id(0)
    ls = ls_r[i]
    le = le_r[i]
    rows = lax.broadcasted_iota(jnp.int32, (_TS, 1), 0)
    m = (rows >= ls) & (rows < le)
    wmat = w_ref[0]
    bvec = b_ref[0, 0]
    xb0 = x0_ref[:, :64].astype(jnp.bfloat16)
    xb1 = x1_ref[:, :64].astype(jnp.bfloat16)
    y0 = jnp.maximum(
        jnp.dot(xb0, wmat, preferred_element_type=jnp.float32) + bvec, 0.0)
    y1 = jnp.maximum(
        jnp.dot(xb1, wmat, preferred_element_type=jnp.float32) + bvec, 0.0)
    o0_ref[:] = jnp.where(m, y0, o0_ref[:])
    o1_ref[:] = jnp.where(m, y1, o1_ref[:])


def _wid():
    return lax.axis_index("s") * 2 + lax.axis_index("c")


def _make_ids_kernel(Fpad):
    E = Fpad // _NW
    mesh = plsc.VectorSubcoreMesh(core_axis_name="c", subcore_axis_name="s")

    @functools.partial(
        pl.kernel, mesh=mesh,
        out_type=jax.ShapeDtypeStruct((Fpad,), jnp.int32),
        scratch_types=[
            pltpu.VMEM((E,), jnp.int32),
            pltpu.VMEM((E,), jnp.int32),
            pltpu.VMEM((E,), jnp.int32),
            pltpu.SemaphoreType.DMA,
        ],
    )
    def ids_kernel(x1_hbm, x2_hbm, f0_hbm, ids_hbm, fv, a1v, a2v, sem):
        base = _wid() * E
        pltpu.sync_copy(f0_hbm.at[pl.ds(base, E)], fv)
        cps = []
        for off, sz in _splits(E):
            sl = pl.ds(off, sz)
            cps.append(pltpu.async_copy(x1_hbm.at[fv.at[sl]], a1v.at[sl], sem))
            cps.append(pltpu.async_copy(x2_hbm.at[fv.at[sl]], a2v.at[sl], sem))
        for cp in cps:
            cp.wait()
        for k in range(E // 16):
            sl = pl.ds(k * 16, 16)
            fv[sl] = a1v[sl] * _MA + a2v[sl]
        pltpu.sync_copy(fv, ids_hbm.at[pl.ds(base, E)])

    return ids_kernel


def _make_gather_rows_kernel(N, Fpad):
    E = Fpad // _NW
    mesh = plsc.VectorSubcoreMesh(core_axis_name="c", subcore_axis_name="s")

    @functools.partial(
        pl.kernel, mesh=mesh,
        out_type=[jax.ShapeDtypeStruct((Fpad, 128), jnp.float32)] * 2,
        scratch_types=[
            pltpu.VMEM((E,), jnp.int32),
            pltpu.VMEM((E,), jnp.int32),
            pltpu.VMEM((E,), jnp.int32),
            pltpu.VMEM((E, 128), jnp.float32),
            pltpu.VMEM((E, 128), jnp.float32),
            pltpu.SemaphoreType.DMA,
        ],
    )
    def gather_kernel(order_hbm, f0_hbm, f1_hbm, nodes_hbm, xs0_hbm, xs1_hbm,
                      ov, g0v, g1v, r0v, r1v, sem):
        base = _wid() * E
        pltpu.sync_copy(order_hbm.at[pl.ds(base, E)], ov)
        cps = []
        for off, sz in _splits(E):
            sl = pl.ds(off, sz)
            cps.append(pltpu.async_copy(f0_hbm.at[ov.at[sl]], g0v.at[sl], sem))
            cps.append(pltpu.async_copy(f1_hbm.at[ov.at[sl]], g1v.at[sl], sem))
        for cp in cps:
            cp.wait()
        cps = []
        for off, sz in _splits(E):
            sl = pl.ds(off, sz)
            cps.append(pltpu.async_copy(nodes_hbm.at[g0v.at[sl]], r0v.at[sl], sem))
            cps.append(pltpu.async_copy(nodes_hbm.at[g1v.at[sl]], r1v.at[sl], sem))
        for cp in cps:
            cp.wait()
        pltpu.sync_copy(r0v, xs0_hbm.at[pl.ds(base, E)])
        pltpu.sync_copy(r1v, xs1_hbm.at[pl.ds(base, E)])

    return gather_kernel


def _make_unsort_kernel(R, F, Fpad):
    E = Fpad // _NW
    cut = F % E
    subs = []
    off = 0
    for lim in ([cut] if 0 < cut < E else []) + [E]:
        while off < lim:
            sz = min(120, lim - off)
            subs.append((off, sz))
            off += sz
    mesh = plsc.VectorSubcoreMesh(core_axis_name="c", subcore_axis_name="s")

    @functools.partial(
        pl.kernel, mesh=mesh,
        out_type=jax.ShapeDtypeStruct((2, F, R), jnp.float32),
        scratch_types=[
            pltpu.VMEM((E,), jnp.int32),
            pltpu.VMEM((E, R), jnp.float32),
            pltpu.VMEM((E, R), jnp.float32),
            pltpu.SemaphoreType.DMA,
        ],
    )
    def unsort_kernel(rank_hbm, ys0_hbm, ys1_hbm, out_hbm, rv, r0v, r1v, sem):
        base = _wid() * E
        pltpu.sync_copy(rank_hbm.at[pl.ds(base, E)], rv)
        for off, sz in subs:
            sl = pl.ds(off, sz)

            @pl.when(base + off < F)
            def _():
                cp0 = pltpu.async_copy(ys0_hbm.at[rv.at[sl]], r0v.at[sl], sem)
                cp1 = pltpu.async_copy(ys1_hbm.at[rv.at[sl]], r1v.at[sl], sem)
                cp0.wait()
                cp1.wait()
                pltpu.sync_copy(r0v.at[sl], out_hbm.at[0, pl.ds(base + off, sz)])
                pltpu.sync_copy(r1v.at[sl], out_hbm.at[1, pl.ds(base + off, sz)])

    return unsort_kernel


def kernel(x, nodes, fact, fact_dim, params, bias):
    del fact_dim
    F = fact.shape[0]
    N = nodes.shape[0]
    L = nodes.shape[1]
    P = params.shape[0]
    R = params.shape[2]
    NT = (F + _TS - 1) // _TS
    Fpad = NT * _TS
    assert Fpad % _NW == 0 and (Fpad // _NW) % 16 == 0
    WMAX = P + NT

    zpad = jnp.zeros((Fpad - F,), jnp.int32)
    f0 = jnp.concatenate([fact[:, 0].astype(jnp.int32), zpad])
    f1 = jnp.concatenate([fact[:, 1].astype(jnp.int32), zpad])
    x1 = x[:, 1].astype(jnp.int32)
    x2 = x[:, 2].astype(jnp.int32)
    nodes_pad = jnp.concatenate(
        [nodes, jnp.zeros((N, 128 - L), jnp.float32)], axis=1)

    ids_p = _make_ids_kernel(Fpad)(x1, x2, f0)

    iota = jnp.arange(Fpad, dtype=jnp.int32)
    sorted_ids, order = lax.sort_key_val(ids_p, iota)
    prev = jnp.concatenate([jnp.full((1,), -1, jnp.int32), sorted_ids[:-1]])
    flags = (sorted_ids != prev) | (iota % _TS == 0)
    pos = jnp.cumsum(flags.astype(jnp.int32)) - 1
    starts = (jnp.full((WMAX,), Fpad, jnp.int32)
              .at[jnp.where(flags, pos, WMAX)].set(iota, mode="drop"))
    ends = jnp.concatenate([starts[1:], jnp.full((1,), Fpad, jnp.int32)])
    tile_w = jnp.minimum(starts // _TS, NT - 1).astype(jnp.int32)
    ls_w = (starts - tile_w * _TS).astype(jnp.int32)
    le_w = (jnp.minimum(ends, (tile_w + 1) * _TS) - tile_w * _TS).astype(jnp.int32)
    id_w = sorted_ids[jnp.minimum(starts, Fpad - 1)]
    rank = jnp.zeros((Fpad,), jnp.int32).at[order].set(iota)

    xs0, xs1 = _make_gather_rows_kernel(N, Fpad)(order, f0, f1, nodes_pad)

    grid_spec = pltpu.PrefetchScalarGridSpec(
        num_scalar_prefetch=4,
        grid=(WMAX,),
        in_specs=[
            pl.BlockSpec((_TS, 128), lambda w, t, i, s, e: (t[w], 0)),
            pl.BlockSpec((_TS, 128), lambda w, t, i, s, e: (t[w], 0)),
            pl.BlockSpec((1, L, R), lambda w, t, i, s, e: (i[w], 0, 0)),
            pl.BlockSpec((1, 1, R), lambda w, t, i, s, e: (i[w], 0, 0)),
        ],
        out_specs=[
            pl.BlockSpec((_TS, R), lambda w, t, i, s, e: (t[w], 0)),
            pl.BlockSpec((_TS, R), lambda w, t, i, s, e: (t[w], 0)),
        ],
    )
    ys0, ys1 = pl.pallas_call(
        _seg_mm_kernel,
        grid_spec=grid_spec,
        out_shape=[jax.ShapeDtypeStruct((Fpad, R), jnp.float32)] * 2,
    )(tile_w, id_w, ls_w, le_w, xs0, xs1,
      params.astype(jnp.bfloat16), bias)

    return _make_unsort_kernel(R, F, Fpad)(rank, ys0, ys1)
